# Initial kernel scaffold; baseline (speedup 1.0000x reference)
import jax
import jax.numpy as jnp
from jax.experimental import pallas as pl


def kernel(x_car, edge_index_truck, edge_index_car, edge_weight_car, src, dst, W_truck, b_truck, W_car, b_car, lin_W, lin_b):
    raise NotImplementedError("write your pallas kernel here")


# speedup vs baseline: 24.5830x; 24.5830x over previous
import functools

import jax
import jax.numpy as jnp
from jax import lax
from jax.experimental import pallas as pl
from jax.experimental.pallas import tpu as pltpu
from jax.experimental.pallas import tpu_sc as plsc

N = 50000
NC = 2
NS = 16
L = 16
NPAD = 50176
NPT = NPAD // NS
E = 800000
EPAD = 800768
EPT = EPAD // NS
CHUNK = 128
NCHUNKS = EPT // CHUNK
Q = 4096
QPW = Q // (NC * NS)
QCH = QPW // L
PADNODE = N

_f32 = jnp.float32
_i32 = jnp.int32


def _fill(ref, n, val):
    def body(i, _):
        ref[pl.ds(i * L, L)] = jnp.full((L,), val, _f32)
        return 0
    lax.fori_loop(0, n // L, body, 0)


def _rsqrt16(x):
    s = 0.5 * (x + 1.0)
    for _ in range(15):
        s = 0.5 * (s + x / s)
    return 1.0 / s


def _conv_body(row_t, col_t, row_c, col_c, ew, x0, x1, x2,
               s_out, z0_out, z1_out, z2_out,
               sh_deg, sh_dinv, sh_a0, sh_a1, sh_a2,
               sh_q0, sh_q1, sh_q2,
               idx0, idx1, b_ew, g0, g1, g2, b_pe,
               nb0, nb1, nb2, ones_b):
    cid = lax.axis_index("c")
    tid = lax.axis_index("s")
    ebase = tid * EPT
    nsl = pl.ds(tid * NPT, NPT)

    _fill(nb0, NPT, 0.0)
    _fill(ones_b, CHUNK, 1.0)

    pltpu.sync_copy(nb0, sh_deg.at[nsl])
    pltpu.sync_copy(nb0, sh_a0.at[nsl])

    @pl.when(cid == 1)
    def _():
        pltpu.sync_copy(nb0, sh_a1.at[nsl])
        pltpu.sync_copy(nb0, sh_a2.at[nsl])

    plsc.subcore_barrier()

    @pl.when(cid == 0)
    def _():
        def bch(j, _):
            pltpu.sync_copy(col_t.at[pl.ds(ebase + j * CHUNK, CHUNK)], idx0)
            pltpu.sync_copy(ones_b, sh_deg.at[idx0], add=True)
            return 0
        lax.fori_loop(0, NCHUNKS, bch, 0)

    @pl.when(cid == 1)
    def _():
        def bch(j, _):
            pltpu.sync_copy(col_c.at[pl.ds(ebase + j * CHUNK, CHUNK)], idx0)
            pltpu.sync_copy(ew.at[pl.ds(ebase + j * CHUNK, CHUNK)], b_ew)
            pltpu.sync_copy(b_ew, sh_deg.at[idx0], add=True)
            return 0
        lax.fori_loop(0, NCHUNKS, bch, 0)

    plsc.subcore_barrier()

    pltpu.sync_copy(sh_deg.at[nsl], nb0)

    def dinv_body(i, _):
        sl = pl.ds(i * L, L)
        nb1[sl] = _rsqrt16(nb0[sl] + 1.0)
        return 0
    lax.fori_loop(0, NPT // L, dinv_body, 0)
    pltpu.sync_copy(nb1, sh_dinv.at[nsl])

    @pl.when(cid == 1)
    def _():
        for xk, shq in ((x0, sh_q0), (x1, sh_q1), (x2, sh_q2)):
            pltpu.sync_copy(xk.at[nsl], nb0)

            def q_body(i, _):
                sl = pl.ds(i * L, L)
                nb0[sl] = nb0[sl] * nb1[sl]
                return 0
            lax.fori_loop(0, NPT // L, q_body, 0)
            pltpu.sync_copy(nb0, shq.at[nsl])

    plsc.subcore_barrier()

    @pl.when(cid == 0)
    def _():
        def bch(j, _):
            esl = pl.ds(ebase + j * CHUNK, CHUNK)
            pltpu.sync_copy(row_t.at[esl], idx1)
            pltpu.sync_copy(col_t.at[esl], idx0)
            pltpu.sync_copy(sh_dinv.at[idx1], g0)
            pltpu.sync_copy(g0, sh_a0.at[idx0], add=True)
            return 0
        lax.fori_loop(0, NCHUNKS, bch, 0)

    @pl.when(cid == 1)
    def _():
        def bch(j, _):
            esl = pl.ds(ebase + j * CHUNK, CHUNK)
            pltpu.sync_copy(row_c.at[esl], idx1)
            pltpu.sync_copy(col_c.at[esl], idx0)
            pltpu.sync_copy(ew.at[esl], b_ew)
            pltpu.sync_copy(sh_dinv.at[idx0], b_pe)
            pltpu.sync_copy(sh_q0.at[idx1], g0)
            pltpu.sync_copy(sh_q1.at[idx1], g1)
            pltpu.sync_copy(sh_q2.at[idx1], g2)

            def mul_body(i, _):
                sl = pl.ds(i * L, L)
                pe = b_ew[sl] * b_pe[sl]
                g0[sl] = g0[sl] * pe
                g1[sl] = g1[sl] * pe
                g2[sl] = g2[sl] * pe
                return 0
            lax.fori_loop(0, CHUNK // L, mul_body, 0)

            pltpu.sync_copy(g0, sh_a0.at[idx0], add=True)
            pltpu.sync_copy(g1, sh_a1.at[idx0], add=True)
            pltpu.sync_copy(g2, sh_a2.at[idx0], add=True)
            return 0
        lax.fori_loop(0, NCHUNKS, bch, 0)

    plsc.subcore_barrier()

    @pl.when(cid == 0)
    def _():
        pltpu.sync_copy(sh_dinv.at[nsl], nb0)
        pltpu.sync_copy(sh_a0.at[nsl], nb1)

        def fin_body(i, _):
            sl = pl.ds(i * L, L)
            d = nb0[sl]
            nb1[sl] = d * (d + nb1[sl])
            return 0
        lax.fori_loop(0, NPT // L, fin_body, 0)
        pltpu.sync_copy(nb1, s_out.at[nsl])

    @pl.when(cid == 1)
    def _():
        pltpu.sync_copy(sh_dinv.at[nsl], nb2)
        for shq, sha, zout in ((sh_q0, sh_a0, z0_out),
                               (sh_q1, sh_a1, z1_out),
                               (sh_q2, sh_a2, z2_out)):
            pltpu.sync_copy(sha.at[nsl], nb0)
            pltpu.sync_copy(shq.at[nsl], nb1)

            def fin_body(i, _):
                sl = pl.ds(i * L, L)
                nb0[sl] = nb0[sl] + nb2[sl] * nb1[sl]
                return 0
            lax.fori_loop(0, NPT // L, fin_body, 0)
            pltpu.sync_copy(nb0, zout.at[nsl])


def _score_body(s_tab, z0_tab, z1_tab, z2_tab, srcq, dstq, gx,
                out,
                i_src, i_dst, it_b, ic_b,
                ss, sd, z0s, z1s, z2s, z0d, z1d, z2d,
                gbuf, outb):
    cid = lax.axis_index("c")
    tid = lax.axis_index("s")
    wid = tid * NC + cid
    qsl = pl.ds(wid * QPW, QPW)

    pltpu.sync_copy(gx, gbuf)
    pltpu.sync_copy(srcq.at[qsl], i_src)
    pltpu.sync_copy(dstq.at[qsl], i_dst)

    zero16 = jnp.zeros((L,), _i32)

    def split(idx_ref):
        def body(i, _):
            sl = pl.ds(i * L, L)
            v = idx_ref[sl]
            lt = v < N
            it_b[sl] = lax.select(lt, v, zero16)
            ic_b[sl] = lax.select(lt, zero16, v - N)
            return 0
        lax.fori_loop(0, QCH, body, 0)

    split(i_src)
    pltpu.sync_copy(s_tab.at[it_b], ss)
    pltpu.sync_copy(z0_tab.at[ic_b], z0s)
    pltpu.sync_copy(z1_tab.at[ic_b], z1s)
    pltpu.sync_copy(z2_tab.at[ic_b], z2s)

    split(i_dst)
    pltpu.sync_copy(s_tab.at[it_b], sd)
    pltpu.sync_copy(z0_tab.at[ic_b], z0d)
    pltpu.sync_copy(z1_tab.at[ic_b], z1d)
    pltpu.sync_copy(z2_tab.at[ic_b], z2d)

    zf = jnp.zeros((L,), _f32)
    of = jnp.ones((L,), _f32)

    def score_body(i, _):
        sl = pl.ds(i * L, L)
        lt_s = i_src[sl] < N
        lt_d = i_dst[sl] < N
        fs = lax.select(lt_s, of, zf)
        fd = lax.select(lt_d, of, zf)
        u = (lax.select(lt_s, ss[sl], zf),
             lax.select(lt_s, zf, z0s[sl]),
             lax.select(lt_s, zf, z1s[sl]),
             lax.select(lt_s, zf, z2s[sl]),
             fs, 1.0 - fs)
        w = (lax.select(lt_d, sd[sl], zf),
             lax.select(lt_d, zf, z0d[sl]),
             lax.select(lt_d, zf, z1d[sl]),
             lax.select(lt_d, zf, z2d[sl]),
             fd, 1.0 - fd)
        acc = zf
        for a in range(6):
            t = zf
            for b in range(6):
                t = t + gbuf[a * 6 + b, :] * w[b]
            acc = acc + u[a] * t
        outb[sl] = acc
        return 0
    lax.fori_loop(0, QCH, score_body, 0)

    pltpu.sync_copy(outb, out.at[qsl])


_mesh = plsc.VectorSubcoreMesh(core_axis_name="c", subcore_axis_name="s",
                               num_cores=NC, num_subcores=NS)

_conv_kernel = functools.partial(
    pl.kernel,
    out_type=[jax.ShapeDtypeStruct((NPAD,), _f32)] * 4,
    mesh=_mesh,
    scratch_types=[pltpu.VMEM_SHARED((NPAD,), _f32)] * 8
    + [pltpu.VMEM((CHUNK,), _i32)] * 2
    + [pltpu.VMEM((CHUNK,), _f32)] * 5
    + [pltpu.VMEM((NPT,), _f32)] * 3
    + [pltpu.VMEM((CHUNK,), _f32)],
)(_conv_body)

_score_kernel = functools.partial(
    pl.kernel,
    out_type=jax.ShapeDtypeStruct((Q,), _f32),
    mesh=_mesh,
    scratch_types=[pltpu.VMEM((QPW,), _i32)] * 4
    + [pltpu.VMEM((QPW,), _f32)] * 8
    + [pltpu.VMEM((36, L), _f32)]
    + [pltpu.VMEM((QPW,), _f32)],
)(_score_body)


def kernel(x_car, edge_index_truck, edge_index_car, edge_weight_car, src, dst,
           W_truck, b_truck, W_car, b_car, lin_W, lin_b):
    f32 = _f32
    ei_t = edge_index_truck.astype(_i32)
    ei_c = edge_index_car.astype(_i32)
    pad = jnp.full((EPAD - E,), PADNODE, _i32)
    row_t = jnp.concatenate([ei_t[0], pad])
    col_t = jnp.concatenate([ei_t[1], pad])
    row_c = jnp.concatenate([ei_c[0], pad])
    col_c = jnp.concatenate([ei_c[1], pad])
    ew = jnp.concatenate([edge_weight_car.astype(f32),
                          jnp.zeros((EPAD - E,), f32)])
    xp = jnp.zeros((3, NPAD), f32).at[:, :N].set(x_car.astype(f32).T)

    s_tab, z0_tab, z1_tab, z2_tab = _conv_kernel(
        row_t, col_t, row_c, col_c, ew, xp[0], xp[1], xp[2])

    M = jnp.concatenate([W_truck.T, W_car.T,
                         b_truck[:, None], b_car[:, None]], axis=1)
    G = M.T @ (lin_W * M)
    G = G.at[4:, 4:].add(lin_b[0])
    gx = jnp.broadcast_to(G.reshape(36, 1), (36, L)).astype(f32)

    scores = _score_kernel(s_tab, z0_tab, z1_tab, z2_tab,
                           src.astype(_i32), dst.astype(_i32), gx)
    return scores[:, None]


# speedup vs baseline: 116.6039x; 4.7433x over previous
import functools

import jax
import jax.numpy as jnp
from jax import lax
from jax.experimental import pallas as pl
from jax.experimental.pallas import tpu as pltpu
from jax.experimental.pallas import tpu_sc as plsc

N = 50000
NC = 2
NS = 16
L = 16
NPAD = 50176
NPT = NPAD // NS
E = 800000
EPAD = 802816
EPT = EPAD // NS
CHUNK = 128
SUB = 56
NSUP = EPT // (SUB * CHUNK)
EROWS = EPAD // CHUNK
Q = 4096
QPW = Q // (NC * NS)
QCH = QPW // L
PADNODE = N
WROWS = 456

_f32 = jnp.float32
_i32 = jnp.int32


def _fill(ref, n, val):
    def body(i, _):
        ref[pl.ds(i * L, L)] = jnp.full((L,), val, _f32)
        return 0
    lax.fori_loop(0, n // L, body, 0)


def _rsqrt16(x):
    s = 0.5 * (x + 1.0)
    for _ in range(15):
        s = 0.5 * (s + x / s)
    return 1.0 / s


def _conv_body(row_t, col_t, row_c, col_c, ew, x0, x1, x2,
               s_out, z0_out, z1_out, z2_out,
               sh_deg, sh_dinv, sh_a0, sh_a1, sh_a2,
               sh_q0, sh_q1, sh_q2,
               idx0, idx1, b_ew, g0, g1, g2, b_pe,
               nb0, nb1, nb2, ones_b,
               lsem, gsem, ssem):
    cid = lax.axis_index("c")
    tid = lax.axis_index("s")
    rbase = tid * SUB * NSUP
    nsl = pl.ds(tid * NPT, NPT)

    _fill(nb0, NPT, 0.0)
    _fill(ones_b, CHUNK, 1.0)

    pltpu.sync_copy(nb0, sh_deg.at[nsl])
    pltpu.sync_copy(nb0, sh_a0.at[nsl])

    @pl.when(cid == 1)
    def _():
        pltpu.sync_copy(nb0, sh_a1.at[nsl])
        pltpu.sync_copy(nb0, sh_a2.at[nsl])

    plsc.subcore_barrier()

    @pl.when(cid == 0)
    def _():
        def sup(sp, _):
            rsl = pl.ds(rbase + sp * SUB, SUB)
            pltpu.async_copy(col_t.at[rsl], idx0, lsem).wait()

            def fire(j, _):
                pltpu.async_copy(ones_b, sh_deg.at[idx0.at[j]], ssem,
                                 add=True)
                return 0
            lax.fori_loop(0, SUB, fire, 0)

            def drain(j, _):
                pltpu.make_async_copy(ones_b, sh_deg.at[idx0.at[0]],
                                      ssem).wait()
                return 0
            lax.fori_loop(0, SUB, drain, 0)
            return 0
        lax.fori_loop(0, NSUP, sup, 0)

    @pl.when(cid == 1)
    def _():
        def sup(sp, _):
            rsl = pl.ds(rbase + sp * SUB, SUB)
            c1 = pltpu.async_copy(col_c.at[rsl], idx0, lsem)
            c2 = pltpu.async_copy(ew.at[rsl], b_ew, lsem)
            c1.wait()
            c2.wait()

            def fire(j, _):
                pltpu.async_copy(b_ew.at[j], sh_deg.at[idx0.at[j]], ssem,
                                 add=True)
                return 0
            lax.fori_loop(0, SUB, fire, 0)

            def drain(j, _):
                pltpu.make_async_copy(ones_b, sh_deg.at[idx0.at[0]],
                                      ssem).wait()
                return 0
            lax.fori_loop(0, SUB, drain, 0)
            return 0
        lax.fori_loop(0, NSUP, sup, 0)

    plsc.subcore_barrier()

    pltpu.sync_copy(sh_deg.at[nsl], nb0)

    def dinv_body(i, _):
        sl = pl.ds(i * L, L)
        nb1[sl] = _rsqrt16(nb0[sl] + 1.0)
        return 0
    lax.fori_loop(0, NPT // L, dinv_body, 0)
    pltpu.sync_copy(nb1, sh_dinv.at[nsl])

    @pl.when(cid == 1)
    def _():
        for xk, shq in ((x0, sh_q0), (x1, sh_q1), (x2, sh_q2)):
            pltpu.sync_copy(xk.at[nsl], nb0)

            def q_body(i, _):
                sl = pl.ds(i * L, L)
                nb0[sl] = nb0[sl] * nb1[sl]
                return 0
            lax.fori_loop(0, NPT // L, q_body, 0)
            pltpu.sync_copy(nb0, shq.at[nsl])

    plsc.subcore_barrier()

    @pl.when(cid == 0)
    def _():
        def sup(sp, _):
            rsl = pl.ds(rbase + sp * SUB, SUB)
            c1 = pltpu.async_copy(row_t.at[rsl], idx1, lsem)
            c2 = pltpu.async_copy(col_t.at[rsl], idx0, lsem)
            c1.wait()
            c2.wait()

            def gfire(j, _):
                pltpu.async_copy(sh_dinv.at[idx1.at[j]], g0.at[j], gsem)
                return 0
            lax.fori_loop(0, SUB, gfire, 0)

            def gdrain(j, _):
                pltpu.make_async_copy(sh_dinv.at[idx1.at[0]], g0.at[0],
                                      gsem).wait()
                return 0
            lax.fori_loop(0, SUB, gdrain, 0)

            def fire(j, _):
                pltpu.async_copy(g0.at[j], sh_a0.at[idx0.at[j]], ssem,
                                 add=True)
                return 0
            lax.fori_loop(0, SUB, fire, 0)

            def drain(j, _):
                pltpu.make_async_copy(g0.at[0], sh_a0.at[idx0.at[0]],
                                      ssem).wait()
                return 0
            lax.fori_loop(0, SUB, drain, 0)
            return 0
        lax.fori_loop(0, NSUP, sup, 0)

    @pl.when(cid == 1)
    def _():
        def sup(sp, _):
            rsl = pl.ds(rbase + sp * SUB, SUB)
            c1 = pltpu.async_copy(row_c.at[rsl], idx1, lsem)
            c2 = pltpu.async_copy(col_c.at[rsl], idx0, lsem)
            c3 = pltpu.async_copy(ew.at[rsl], b_ew, lsem)
            c1.wait()
            c2.wait()
            c3.wait()

            def gfire(j, _):
                pltpu.async_copy(sh_dinv.at[idx0.at[j]], b_pe.at[j], gsem)
                pltpu.async_copy(sh_q0.at[idx1.at[j]], g0.at[j], gsem)
                pltpu.async_copy(sh_q1.at[idx1.at[j]], g1.at[j], gsem)
                pltpu.async_copy(sh_q2.at[idx1.at[j]], g2.at[j], gsem)
                return 0
            lax.fori_loop(0, SUB, gfire, 0)

            def gdrain(j, _):
                for _k in range(4):
                    pltpu.make_async_copy(sh_dinv.at[idx0.at[0]], b_pe.at[0],
                                          gsem).wait()
                return 0
            lax.fori_loop(0, SUB, gdrain, 0)

            def mul_body(j, _):
                def mul_inner(i, _):
                    sl = pl.ds(i * L, L)
                    pe = b_ew[j, sl] * b_pe[j, sl]
                    g0[j, sl] = g0[j, sl] * pe
                    g1[j, sl] = g1[j, sl] * pe
                    g2[j, sl] = g2[j, sl] * pe
                    return 0
                lax.fori_loop(0, CHUNK // L, mul_inner, 0)
                return 0
            lax.fori_loop(0, SUB, mul_body, 0)

            def fire(j, _):
                pltpu.async_copy(g0.at[j], sh_a0.at[idx0.at[j]], ssem,
                                 add=True)
                pltpu.async_copy(g1.at[j], sh_a1.at[idx0.at[j]], ssem,
                                 add=True)
                pltpu.async_copy(g2.at[j], sh_a2.at[idx0.at[j]], ssem,
                                 add=True)
                return 0
            lax.fori_loop(0, SUB, fire, 0)

            def drain(j, _):
                for _k in range(3):
                    pltpu.make_async_copy(g0.at[0], sh_a0.at[idx0.at[0]],
                                          ssem).wait()
                return 0
            lax.fori_loop(0, SUB, drain, 0)
            return 0
        lax.fori_loop(0, NSUP, sup, 0)

    plsc.subcore_barrier()

    @pl.when(cid == 0)
    def _():
        pltpu.sync_copy(sh_dinv.at[nsl], nb0)
        pltpu.sync_copy(sh_a0.at[nsl], nb1)

        def fin_body(i, _):
            sl = pl.ds(i * L, L)
            d = nb0[sl]
            nb1[sl] = d * (d + nb1[sl])
            return 0
        lax.fori_loop(0, NPT // L, fin_body, 0)
        pltpu.sync_copy(nb1, s_out.at[nsl])

    @pl.when(cid == 1)
    def _():
        pltpu.sync_copy(sh_dinv.at[nsl], nb2)
        for shq, sha, zout in ((sh_q0, sh_a0, z0_out),
                               (sh_q1, sh_a1, z1_out),
                               (sh_q2, sh_a2, z2_out)):
            pltpu.sync_copy(sha.at[nsl], nb0)
            pltpu.sync_copy(shq.at[nsl], nb1)

            def fin_body(i, _):
                sl = pl.ds(i * L, L)
                nb0[sl] = nb0[sl] + nb2[sl] * nb1[sl]
                return 0
            lax.fori_loop(0, NPT // L, fin_body, 0)
            pltpu.sync_copy(nb0, zout.at[nsl])


def _score_body(s_tab, z0_tab, z1_tab, z2_tab, srcq, dstq,
                it_s, ic_s, it_d, ic_d, wpack,
                out,
                sh_s, sh_z0, sh_z1, sh_z2,
                i_src, i_dst, it_b, ic_b,
                ss, sd, z0s, z1s, z2s, z0d, z1d, z2d,
                wbuf, outb, nbb):
    cid = lax.axis_index("c")
    tid = lax.axis_index("s")
    wid = tid * NC + cid
    qsl = pl.ds(wid * QPW, QPW)
    nsl = pl.ds(tid * NPT, NPT)

    for tab, sh in ((s_tab, sh_s), (z0_tab, sh_z0),
                    (z1_tab, sh_z1), (z2_tab, sh_z2)):
        pltpu.sync_copy(tab.at[nsl], nbb)
        pltpu.sync_copy(nbb, sh.at[nsl])
    plsc.subcore_barrier()

    pltpu.sync_copy(wpack, wbuf)
    pltpu.sync_copy(srcq.at[qsl], i_src)
    pltpu.sync_copy(dstq.at[qsl], i_dst)

    pltpu.sync_copy(it_s.at[qsl], it_b)
    pltpu.sync_copy(ic_s.at[qsl], ic_b)
    pltpu.sync_copy(sh_s.at[it_b], ss)
    pltpu.sync_copy(sh_z0.at[ic_b], z0s)
    pltpu.sync_copy(sh_z1.at[ic_b], z1s)
    pltpu.sync_copy(sh_z2.at[ic_b], z2s)

    pltpu.sync_copy(it_d.at[qsl], it_b)
    pltpu.sync_copy(ic_d.at[qsl], ic_b)
    pltpu.sync_copy(sh_s.at[it_b], sd)
    pltpu.sync_copy(sh_z0.at[ic_b], z0d)
    pltpu.sync_copy(sh_z1.at[ic_b], z1d)
    pltpu.sync_copy(sh_z2.at[ic_b], z2d)

    def score_body(i, _):
        sl = pl.ds(i * L, L)
        lt_s = i_src[sl] < N
        lt_d = i_dst[sl] < N
        s_s, s_d = ss[sl], sd[sl]
        zs = (z0s[sl], z1s[sl], z2s[sl])
        zd = (z0d[sl], z1d[sl], z2d[sl])

        def kbody(k, acc):
            wt = wbuf[k, :]
            w0 = wbuf[64 + k, :]
            w1 = wbuf[128 + k, :]
            w2 = wbuf[192 + k, :]
            lw = wbuf[256 + k, :]
            bt = wbuf[320 + k, :]
            bc = wbuf[384 + k, :]
            se = jnp.where(lt_s, s_s * wt + bt,
                           zs[0] * w0 + zs[1] * w1 + zs[2] * w2 + bc)
            de = jnp.where(lt_d, s_d * wt + bt,
                           zd[0] * w0 + zd[1] * w1 + zd[2] * w2 + bc)
            p = se * de
            t = p * 65537.0
            pb = t - (t - p)
            return acc + pb * lw

        acc0 = wbuf[448, :]
        outb[sl] = lax.fori_loop(0, 64, kbody, acc0)
        return 0
    lax.fori_loop(0, QCH, score_body, 0)

    pltpu.sync_copy(outb, out.at[qsl])


_mesh = plsc.VectorSubcoreMesh(core_axis_name="c", subcore_axis_name="s",
                               num_cores=NC, num_subcores=NS)

_conv_kernel = functools.partial(
    pl.kernel,
    out_type=[jax.ShapeDtypeStruct((NPAD,), _f32)] * 4,
    mesh=_mesh,
    scratch_types=[pltpu.VMEM_SHARED((NPAD,), _f32)] * 8
    + [pltpu.VMEM((SUB, CHUNK), _i32)] * 2
    + [pltpu.VMEM((SUB, CHUNK), _f32)] * 5
    + [pltpu.VMEM((NPT,), _f32)] * 3
    + [pltpu.VMEM((CHUNK,), _f32)]
    + [pltpu.SemaphoreType.DMA] * 3,
)(_conv_body)

_score_kernel = functools.partial(
    pl.kernel,
    out_type=jax.ShapeDtypeStruct((Q,), _f32),
    mesh=_mesh,
    scratch_types=[pltpu.VMEM_SHARED((NPAD,), _f32)] * 4
    + [pltpu.VMEM((QPW,), _i32)] * 4
    + [pltpu.VMEM((QPW,), _f32)] * 8
    + [pltpu.VMEM((WROWS, L), _f32)]
    + [pltpu.VMEM((QPW,), _f32)]
    + [pltpu.VMEM((NPT,), _f32)],
)(_score_body)


def kernel(x_car, edge_index_truck, edge_index_car, edge_weight_car, src, dst,
           W_truck, b_truck, W_car, b_car, lin_W, lin_b):
    f32 = _f32
    ei_t = edge_index_truck.astype(_i32)
    ei_c = edge_index_car.astype(_i32)
    pad = jnp.full((EPAD - E,), PADNODE, _i32)
    row_t = jnp.concatenate([ei_t[0], pad]).reshape(EROWS, CHUNK)
    col_t = jnp.concatenate([ei_t[1], pad]).reshape(EROWS, CHUNK)
    row_c = jnp.concatenate([ei_c[0], pad]).reshape(EROWS, CHUNK)
    col_c = jnp.concatenate([ei_c[1], pad]).reshape(EROWS, CHUNK)
    ew = jnp.concatenate([edge_weight_car.astype(f32),
                          jnp.zeros((EPAD - E,), f32)]).reshape(EROWS, CHUNK)
    xbf = x_car.astype(jnp.bfloat16).astype(f32)
    xp = jnp.zeros((3, NPAD), f32).at[:, :N].set(xbf.T)

    s_tab, z0_tab, z1_tab, z2_tab = _conv_kernel(
        row_t, col_t, row_c, col_c, ew, xp[0], xp[1], xp[2])

    bf = jnp.bfloat16
    rows = jnp.concatenate([
        W_truck[0].astype(bf).astype(f32),
        W_car[0].astype(bf).astype(f32),
        W_car[1].astype(bf).astype(f32),
        W_car[2].astype(bf).astype(f32),
        lin_W[:, 0].astype(bf).astype(f32),
        b_truck.astype(f32),
        b_car.astype(f32),
        lin_b.astype(f32),
        jnp.zeros((WROWS - 449,), f32),
    ])
    wpack = jnp.broadcast_to(rows[:, None], (WROWS, L))

    srci = src.astype(_i32)
    dsti = dst.astype(_i32)
    lt_s = srci < N
    lt_d = dsti < N
    it_s = jnp.where(lt_s, srci, 0)
    ic_s = jnp.where(lt_s, 0, srci - N)
    it_d = jnp.where(lt_d, dsti, 0)
    ic_d = jnp.where(lt_d, 0, dsti - N)
    scores = _score_kernel(s_tab, z0_tab, z1_tab, z2_tab, srci, dsti,
                           it_s, ic_s, it_d, ic_d, wpack)
    return scores[:, None]
